# baseline (speedup 1.0000x reference)
import jax
import jax.numpy as jnp
from jax import lax
from jax.experimental import pallas as pl
from jax.experimental.pallas import tpu as pltpu
from jax.experimental.pallas import tpu_sc as plsc

_CONF = 0.25
_IOU_THR = 0.45
_MAX_DET = 300
_N_BOXES = 20000

_NT = 16
_L = 16
_CH = 1280
_NV = _CH // _L
_NPAD = _NT * _CH
_OUT_PAD = 320
_XW = _NT * _L
_BIG = 2**30


def _nms_call(x1, y1, x2, y2, s, c):
    mesh = plsc.VectorSubcoreMesh(
        core_axis_name="c", subcore_axis_name="s", num_cores=1,
        num_subcores=_NT)

    def body(x1h, y1h, x2h, y2h, sh, ch, out_h,
             x1v, y1v, x2v, y2v, sv, areav, clsv, stag, allv, outv, shp):
        tid = lax.axis_index("s")
        base = tid * _CH
        lane = lax.broadcasted_iota(jnp.int32, (_L,), 0)
        zerof = jnp.zeros((_L,), jnp.float32)
        neg2 = jnp.full((_L,), -2.0, jnp.float32)
        zero_i = jnp.zeros((_L,), jnp.int32)

        pltpu.sync_copy(x1h.at[pl.ds(base, _CH)], x1v)
        pltpu.sync_copy(y1h.at[pl.ds(base, _CH)], y1v)
        pltpu.sync_copy(x2h.at[pl.ds(base, _CH)], x2v)
        pltpu.sync_copy(y2h.at[pl.ds(base, _CH)], y2v)
        pltpu.sync_copy(sh.at[pl.ds(base, _CH)], sv)
        pltpu.sync_copy(ch.at[pl.ds(base, _CH)], clsv)

        def seed0(ref):
            plsc.store_scatter(ref, [lane], zerof, mask=lane < 1)

        def init_i(i, carry):
            off, bestv, bposv = carry
            sl = pl.ds(i * _L, _L)
            sval = sv[sl]
            bx1 = x1v[sl]
            by1 = y1v[sl]
            bx2 = x2v[sl]
            by2 = y2v[sl]
            bcl = clsv[sl]
            m = sval > _CONF
            pos = off + plsc.cumsum(m.astype(jnp.int32)) - 1
            plsc.store_scatter(sv, [pos], sval, mask=m)
            plsc.store_scatter(x1v, [pos], bx1, mask=m)
            plsc.store_scatter(y1v, [pos], by1, mask=m)
            plsc.store_scatter(x2v, [pos], bx2, mask=m)
            plsc.store_scatter(y2v, [pos], by2, mask=m)
            plsc.store_scatter(clsv, [pos], bcl, mask=m)
            area = (bx2 - bx1) * (by2 - by1)
            plsc.store_scatter(areav, [pos], area, mask=m)
            better = m & (sval > bestv)
            bestv = jnp.where(better, sval, bestv)
            bposv = jnp.where(better, pos, bposv)
            return (off + plsc.all_reduce_population_count(m), bestv, bposv)

        seed0(x1v)
        seed0(y1v)
        seed0(x2v)
        seed0(y2v)
        seed0(areav)
        seed0(clsv)
        offv, bestv0, bposv0 = lax.fori_loop(
            0, _NV, init_i, (zero_i, neg2, zero_i))
        act = jnp.max(offv)
        negones = jnp.full((_L,), -1.0, jnp.float32)
        for k in range(4):
            tpos = offv + k * _L + lane
            plsc.store_scatter(sv, [tpos], negones, mask=tpos < _CH)
        nch = jnp.right_shift(act + 63, 6)

        full15 = jnp.full((_L,), _L - 1, jnp.int32)
        gdn = lax.GatherDimensionNumbers(
            offset_dims=(), collapsed_slice_dims=(0,), start_index_map=(0,))

        def bclast(v):
            return lax.gather(v, full15[:, None], gdn, (1,),
                              mode=lax.GatherScatterMode.PROMISE_IN_BOUNDS)

        def step(t, carry):
            bestv, bposv = carry
            maxs = bclast(plsc.cummax(bestv))
            lidx = -bclast(plsc.cummax(-jnp.where(bestv == maxs, bposv, _BIG)))
            wx1l = plsc.load_gather(x1v, [lidx])
            wy1l = plsc.load_gather(y1v, [lidx])
            wx2l = plsc.load_gather(x2v, [lidx])
            wy2l = plsc.load_gather(y2v, [lidx])
            wal = plsc.load_gather(areav, [lidx])
            wcl = plsc.load_gather(clsv, [lidx])
            row = maxs
            row = jnp.where(lane == 1, wx1l, row)
            row = jnp.where(lane == 2, wy1l, row)
            row = jnp.where(lane == 3, wx2l, row)
            row = jnp.where(lane == 4, wy2l, row)
            row = jnp.where(lane == 5, wal, row)
            row = jnp.where(lane == 6, wcl, row)
            stag[...] = row
            pbuf = jnp.bitwise_and(t, 1) * _XW
            pltpu.sync_copy(stag, shp.at[pl.ds(pbuf + tid * _L, _L)])
            plsc.subcore_barrier()
            pltpu.sync_copy(shp.at[pl.ds(pbuf, _XW)], allv)

            msv = plsc.load_gather(allv, [lane * _L])
            gms = bclast(plsc.cummax(msv))
            wts = -bclast(plsc.cummax(
                -jnp.where(msv == gms, lane, jnp.int32(_L - 1))))
            wb = wts * _L
            wx1 = plsc.load_gather(allv, [wb + 1])
            wy1 = plsc.load_gather(allv, [wb + 2])
            wx2 = plsc.load_gather(allv, [wb + 3])
            wy2 = plsc.load_gather(allv, [wb + 4])
            wa = plsc.load_gather(allv, [wb + 5])
            wcls = plsc.load_gather(allv, [wb + 6])
            validv = gms > 0.0
            vfv = jnp.where(validv, 1.0, 0.0)

            @pl.when(tid == 0)
            def _():
                ov = vfv
                ov = jnp.where(lane == 0, wx1 * vfv, ov)
                ov = jnp.where(lane == 1, wy1 * vfv, ov)
                ov = jnp.where(lane == 2, wx2 * vfv, ov)
                ov = jnp.where(lane == 3, wy2 * vfv, ov)
                ov = jnp.where(lane == 4, jnp.where(validv, wcls, -1.0), ov)
                ov = jnp.where(lane == 5, jnp.where(validv, gms, 0.0), ov)
                plsc.store_scatter(outv, [t * 8 + lane], ov, mask=lane < 7)

            @plsc.parallel_loop(0, nch, 1, unroll=2, carry=(neg2, zero_i))
            def sup_i(i, cc):
                bv, bpv = cc
                for k in range(4):
                    sl = pl.ds(i * 64 + k * _L, _L)
                    ix1 = jnp.maximum(wx1, x1v[sl])
                    iy1 = jnp.maximum(wy1, y1v[sl])
                    ix2 = jnp.minimum(wx2, x2v[sl])
                    iy2 = jnp.minimum(wy2, y2v[sl])
                    inter = (jnp.maximum(ix2 - ix1, 0.0)
                             * jnp.maximum(iy2 - iy1, 0.0))
                    iou = inter / (wa + areav[sl] - inter + 1e-9)
                    s_new = jnp.where(iou > _IOU_THR, -1.0, sv[sl])
                    sv[sl] = s_new
                    pos = i * 64 + k * _L + lane
                    better = s_new > bv
                    bv = jnp.where(better, s_new, bv)
                    bpv = jnp.where(better, pos, bpv)
                return bv, bpv

            return sup_i

        lax.fori_loop(0, _MAX_DET, step, (bestv0, bposv0))

        @pl.when(tid == 0)
        def _():
            pltpu.sync_copy(outv, out_h)

    call = pl.kernel(
        body,
        out_type=jax.ShapeDtypeStruct((_OUT_PAD * 8,), jnp.float32),
        mesh=mesh,
        compiler_params=pltpu.CompilerParams(needs_layout_passes=False),
        scratch_types=[
            pltpu.VMEM((_CH,), jnp.float32),
            pltpu.VMEM((_CH,), jnp.float32),
            pltpu.VMEM((_CH,), jnp.float32),
            pltpu.VMEM((_CH,), jnp.float32),
            pltpu.VMEM((_CH,), jnp.float32),
            pltpu.VMEM((_CH,), jnp.float32),
            pltpu.VMEM((_CH,), jnp.float32),
            pltpu.VMEM((_L,), jnp.float32),
            pltpu.VMEM((_XW,), jnp.float32),
            pltpu.VMEM((_OUT_PAD * 8,), jnp.float32),
            pltpu.VMEM_SHARED((2 * _XW,), jnp.float32),
        ],
    )
    return call(x1, y1, x2, y2, s, c)


def kernel(boxes, scores, classes):
    pad = _NPAD - _N_BOXES
    x1 = jnp.pad(boxes[:, 0], (0, pad))
    y1 = jnp.pad(boxes[:, 1], (0, pad))
    x2 = jnp.pad(boxes[:, 2], (0, pad))
    y2 = jnp.pad(boxes[:, 3], (0, pad))
    s = jnp.pad(scores, (0, pad), constant_values=-1.0)
    c = jnp.pad(classes.astype(jnp.float32), (0, pad))

    out = _nms_call(x1, y1, x2, y2, s, c)
    out = out.reshape(_OUT_PAD, 8)[:_MAX_DET]
    boxes_b = out[:, 0:4]
    labels_b = out[:, 4].astype(jnp.int32)
    scores_b = out[:, 5]
    sel_valid = out[:, 6] > 0.0
    return boxes_b, labels_b, scores_b, sel_valid


# speedup vs baseline: 1.1085x; 1.1085x over previous
import jax
import jax.numpy as jnp
from jax import lax
from jax.experimental import pallas as pl
from jax.experimental.pallas import tpu as pltpu
from jax.experimental.pallas import tpu_sc as plsc

_CONF = 0.25
_IOU_THR = 0.45
_MAX_DET = 300
_N_BOXES = 20000

_NT = 16
_L = 16
_CH = 1280
_NV = _CH // _L
_NPAD = _NT * _CH
_OUT_PAD = 320
_XW = _NT * _L
_BIG = 2**30


def _nms_call(x1, y1, x2, y2, s, c):
    mesh = plsc.VectorSubcoreMesh(
        core_axis_name="c", subcore_axis_name="s", num_cores=1,
        num_subcores=_NT)

    def body(x1h, y1h, x2h, y2h, sh, ch, out_h,
             x1v, y1v, x2v, y2v, sv, areav, clsv, stag, allv, outv, shp):
        tid = lax.axis_index("s")
        base = tid * _CH
        lane = lax.broadcasted_iota(jnp.int32, (_L,), 0)
        zerof = jnp.zeros((_L,), jnp.float32)
        neg2 = jnp.full((_L,), -2.0, jnp.float32)
        zero_i = jnp.zeros((_L,), jnp.int32)

        pltpu.sync_copy(x1h.at[pl.ds(base, _CH)], x1v)
        pltpu.sync_copy(y1h.at[pl.ds(base, _CH)], y1v)
        pltpu.sync_copy(x2h.at[pl.ds(base, _CH)], x2v)
        pltpu.sync_copy(y2h.at[pl.ds(base, _CH)], y2v)
        pltpu.sync_copy(sh.at[pl.ds(base, _CH)], sv)
        pltpu.sync_copy(ch.at[pl.ds(base, _CH)], clsv)

        def seed0(ref):
            plsc.store_scatter(ref, [lane], zerof, mask=lane < 1)

        def init_i(i, carry):
            off, bestv, bposv = carry
            sl = pl.ds(i * _L, _L)
            sval = sv[sl]
            bx1 = x1v[sl]
            by1 = y1v[sl]
            bx2 = x2v[sl]
            by2 = y2v[sl]
            bcl = clsv[sl]
            m = sval > _CONF
            pos = off + plsc.cumsum(m.astype(jnp.int32)) - 1
            plsc.store_scatter(sv, [pos], sval, mask=m)
            plsc.store_scatter(x1v, [pos], bx1, mask=m)
            plsc.store_scatter(y1v, [pos], by1, mask=m)
            plsc.store_scatter(x2v, [pos], bx2, mask=m)
            plsc.store_scatter(y2v, [pos], by2, mask=m)
            plsc.store_scatter(clsv, [pos], bcl, mask=m)
            area = (bx2 - bx1) * (by2 - by1)
            plsc.store_scatter(areav, [pos], area, mask=m)
            better = m & (sval > bestv)
            bestv = jnp.where(better, sval, bestv)
            bposv = jnp.where(better, pos, bposv)
            return (off + plsc.all_reduce_population_count(m), bestv, bposv)

        seed0(x1v)
        seed0(y1v)
        seed0(x2v)
        seed0(y2v)
        seed0(areav)
        seed0(clsv)
        offv, bestv0, bposv0 = lax.fori_loop(
            0, _NV, init_i, (zero_i, neg2, zero_i))
        act = jnp.max(offv)
        negones = jnp.full((_L,), -1.0, jnp.float32)
        for k in range(2):
            tpos = offv + k * _L + lane
            plsc.store_scatter(sv, [tpos], negones, mask=tpos < _CH)
        nch = jnp.right_shift(act + 31, 5)

        full15 = jnp.full((_L,), _L - 1, jnp.int32)
        gdn = lax.GatherDimensionNumbers(
            offset_dims=(), collapsed_slice_dims=(0,), start_index_map=(0,))

        def bclast(v):
            return lax.gather(v, full15[:, None], gdn, (1,),
                              mode=lax.GatherScatterMode.PROMISE_IN_BOUNDS)

        def step(t, carry):
            bestv, bposv = carry
            maxs = bclast(plsc.cummax(bestv))
            lidx = -bclast(plsc.cummax(-jnp.where(bestv == maxs, bposv, _BIG)))
            wx1l = plsc.load_gather(x1v, [lidx])
            wy1l = plsc.load_gather(y1v, [lidx])
            wx2l = plsc.load_gather(x2v, [lidx])
            wy2l = plsc.load_gather(y2v, [lidx])
            wal = plsc.load_gather(areav, [lidx])
            wcl = plsc.load_gather(clsv, [lidx])
            row = maxs
            row = jnp.where(lane == 1, wx1l, row)
            row = jnp.where(lane == 2, wy1l, row)
            row = jnp.where(lane == 3, wx2l, row)
            row = jnp.where(lane == 4, wy2l, row)
            row = jnp.where(lane == 5, wal, row)
            row = jnp.where(lane == 6, wcl, row)
            stag[...] = row
            pbuf = jnp.bitwise_and(t, 1) * _XW
            pltpu.sync_copy(stag, shp.at[pl.ds(pbuf + tid * _L, _L)])
            plsc.subcore_barrier()
            pltpu.sync_copy(shp.at[pl.ds(pbuf, _XW)], allv)

            msv = plsc.load_gather(allv, [lane * _L])
            gms = bclast(plsc.cummax(msv))
            wts = -bclast(plsc.cummax(
                -jnp.where(msv == gms, lane, jnp.int32(_L - 1))))
            wb = wts * _L
            wx1 = plsc.load_gather(allv, [wb + 1])
            wy1 = plsc.load_gather(allv, [wb + 2])
            wx2 = plsc.load_gather(allv, [wb + 3])
            wy2 = plsc.load_gather(allv, [wb + 4])
            wa = plsc.load_gather(allv, [wb + 5])
            wcls = plsc.load_gather(allv, [wb + 6])
            validv = gms > 0.0
            vfv = jnp.where(validv, 1.0, 0.0)

            @pl.when(tid == 0)
            def _():
                ov = vfv
                ov = jnp.where(lane == 0, wx1 * vfv, ov)
                ov = jnp.where(lane == 1, wy1 * vfv, ov)
                ov = jnp.where(lane == 2, wx2 * vfv, ov)
                ov = jnp.where(lane == 3, wy2 * vfv, ov)
                ov = jnp.where(lane == 4, jnp.where(validv, wcls, -1.0), ov)
                ov = jnp.where(lane == 5, jnp.where(validv, gms, 0.0), ov)
                plsc.store_scatter(outv, [t * 8 + lane], ov, mask=lane < 7)

            @plsc.parallel_loop(0, nch, 1, carry=(neg2, zero_i))
            def sup_i(i, cc):
                bv, bpv = cc
                for k in range(2):
                    sl = pl.ds(i * 32 + k * _L, _L)
                    ix1 = jnp.maximum(wx1, x1v[sl])
                    iy1 = jnp.maximum(wy1, y1v[sl])
                    ix2 = jnp.minimum(wx2, x2v[sl])
                    iy2 = jnp.minimum(wy2, y2v[sl])
                    inter = (jnp.maximum(ix2 - ix1, 0.0)
                             * jnp.maximum(iy2 - iy1, 0.0))
                    iou = inter / (wa + areav[sl] - inter + 1e-9)
                    s_new = jnp.where(iou > _IOU_THR, -1.0, sv[sl])
                    sv[sl] = s_new
                    pos = i * 32 + k * _L + lane
                    better = s_new > bv
                    bv = jnp.where(better, s_new, bv)
                    bpv = jnp.where(better, pos, bpv)
                return bv, bpv

            return sup_i

        lax.fori_loop(0, _MAX_DET, step, (bestv0, bposv0))

        @pl.when(tid == 0)
        def _():
            pltpu.sync_copy(outv, out_h)

    call = pl.kernel(
        body,
        out_type=jax.ShapeDtypeStruct((_OUT_PAD * 8,), jnp.float32),
        mesh=mesh,
        compiler_params=pltpu.CompilerParams(needs_layout_passes=False),
        scratch_types=[
            pltpu.VMEM((_CH,), jnp.float32),
            pltpu.VMEM((_CH,), jnp.float32),
            pltpu.VMEM((_CH,), jnp.float32),
            pltpu.VMEM((_CH,), jnp.float32),
            pltpu.VMEM((_CH,), jnp.float32),
            pltpu.VMEM((_CH,), jnp.float32),
            pltpu.VMEM((_CH,), jnp.float32),
            pltpu.VMEM((_L,), jnp.float32),
            pltpu.VMEM((_XW,), jnp.float32),
            pltpu.VMEM((_OUT_PAD * 8,), jnp.float32),
            pltpu.VMEM_SHARED((2 * _XW,), jnp.float32),
        ],
    )
    return call(x1, y1, x2, y2, s, c)


def kernel(boxes, scores, classes):
    pad = _NPAD - _N_BOXES
    x1 = jnp.pad(boxes[:, 0], (0, pad))
    y1 = jnp.pad(boxes[:, 1], (0, pad))
    x2 = jnp.pad(boxes[:, 2], (0, pad))
    y2 = jnp.pad(boxes[:, 3], (0, pad))
    s = jnp.pad(scores, (0, pad), constant_values=-1.0)
    c = jnp.pad(classes.astype(jnp.float32), (0, pad))

    out = _nms_call(x1, y1, x2, y2, s, c)
    out = out.reshape(_OUT_PAD, 8)[:_MAX_DET]
    boxes_b = out[:, 0:4]
    labels_b = out[:, 4].astype(jnp.int32)
    scores_b = out[:, 5]
    sel_valid = out[:, 6] > 0.0
    return boxes_b, labels_b, scores_b, sel_valid


# speedup vs baseline: 1.1556x; 1.0425x over previous
import jax
import jax.numpy as jnp
from jax import lax
from jax.experimental import pallas as pl
from jax.experimental.pallas import tpu as pltpu
from jax.experimental.pallas import tpu_sc as plsc

_CONF = 0.25
_IOU_THR = 0.45
_MAX_DET = 300
_N_BOXES = 20000

_NT = 16
_L = 16
_CH = 1280
_NV = _CH // _L
_NPAD = _NT * _CH
_OUT_PAD = 320
_XW = _NT * _L
_BIG = 2**30


def _nms_call(x1, y1, x2, y2, s, c):
    mesh = plsc.VectorSubcoreMesh(
        core_axis_name="c", subcore_axis_name="s", num_cores=1,
        num_subcores=_NT)

    def body(x1h, y1h, x2h, y2h, sh, ch, out_h,
             x1v, y1v, x2v, y2v, sv, areav, clsv, stag, allv, outv, shp):
        tid = lax.axis_index("s")
        base = tid * _CH
        lane = lax.broadcasted_iota(jnp.int32, (_L,), 0)
        zerof = jnp.zeros((_L,), jnp.float32)
        neg2 = jnp.full((_L,), -2.0, jnp.float32)
        zero_i = jnp.zeros((_L,), jnp.int32)

        pltpu.sync_copy(x1h.at[pl.ds(base, _CH)], x1v)
        pltpu.sync_copy(y1h.at[pl.ds(base, _CH)], y1v)
        pltpu.sync_copy(x2h.at[pl.ds(base, _CH)], x2v)
        pltpu.sync_copy(y2h.at[pl.ds(base, _CH)], y2v)
        pltpu.sync_copy(sh.at[pl.ds(base, _CH)], sv)
        pltpu.sync_copy(ch.at[pl.ds(base, _CH)], clsv)

        def seed0(ref):
            plsc.store_scatter(ref, [lane], zerof, mask=lane < 1)

        def init_i(i, carry):
            off, bestv, bposv = carry
            sl = pl.ds(i * _L, _L)
            sval = sv[sl]
            bx1 = x1v[sl]
            by1 = y1v[sl]
            bx2 = x2v[sl]
            by2 = y2v[sl]
            bcl = clsv[sl]
            m = sval > _CONF
            pos = off + plsc.cumsum(m.astype(jnp.int32)) - 1
            plsc.store_scatter(sv, [pos], sval, mask=m)
            plsc.store_scatter(x1v, [pos], bx1, mask=m)
            plsc.store_scatter(y1v, [pos], by1, mask=m)
            plsc.store_scatter(x2v, [pos], bx2, mask=m)
            plsc.store_scatter(y2v, [pos], by2, mask=m)
            plsc.store_scatter(clsv, [pos], bcl, mask=m)
            area = (bx2 - bx1) * (by2 - by1)
            plsc.store_scatter(areav, [pos], area, mask=m)
            better = m & (sval > bestv)
            bestv = jnp.where(better, sval, bestv)
            bposv = jnp.where(better, pos, bposv)
            return (off + plsc.all_reduce_population_count(m), bestv, bposv)

        seed0(x1v)
        seed0(y1v)
        seed0(x2v)
        seed0(y2v)
        seed0(areav)
        seed0(clsv)
        offv, bestv0, bposv0 = lax.fori_loop(
            0, _NV, init_i, (zero_i, neg2, zero_i))
        act = jnp.max(offv)
        negones = jnp.full((_L,), -1.0, jnp.float32)
        for k in range(4):
            tpos = offv + k * _L + lane
            plsc.store_scatter(sv, [tpos], negones, mask=tpos < _CH)
        nch = jnp.right_shift(act + 63, 6)

        full15 = jnp.full((_L,), _L - 1, jnp.int32)
        gdn = lax.GatherDimensionNumbers(
            offset_dims=(), collapsed_slice_dims=(0,), start_index_map=(0,))

        def bclast(v):
            return lax.gather(v, full15[:, None], gdn, (1,),
                              mode=lax.GatherScatterMode.PROMISE_IN_BOUNDS)

        def step(t, carry):
            bestv, bposv = carry
            maxs = bclast(plsc.cummax(bestv))
            lidx = -bclast(plsc.cummax(-jnp.where(bestv == maxs, bposv, _BIG)))
            wx1l = plsc.load_gather(x1v, [lidx])
            wy1l = plsc.load_gather(y1v, [lidx])
            wx2l = plsc.load_gather(x2v, [lidx])
            wy2l = plsc.load_gather(y2v, [lidx])
            wal = plsc.load_gather(areav, [lidx])
            wcl = plsc.load_gather(clsv, [lidx])
            row = maxs
            row = jnp.where(lane == 1, wx1l, row)
            row = jnp.where(lane == 2, wy1l, row)
            row = jnp.where(lane == 3, wx2l, row)
            row = jnp.where(lane == 4, wy2l, row)
            row = jnp.where(lane == 5, wal, row)
            row = jnp.where(lane == 6, wcl, row)
            stag[...] = row
            pbuf = jnp.bitwise_and(t, 1) * _XW
            pltpu.sync_copy(stag, shp.at[pl.ds(pbuf + tid * _L, _L)])
            plsc.subcore_barrier()
            pltpu.sync_copy(shp.at[pl.ds(pbuf, _XW)], allv)

            msv = plsc.load_gather(allv, [lane * _L])
            gms = bclast(plsc.cummax(msv))
            wts = -bclast(plsc.cummax(
                -jnp.where(msv == gms, lane, jnp.int32(_L - 1))))
            wb = wts * _L
            wx1 = plsc.load_gather(allv, [wb + 1])
            wy1 = plsc.load_gather(allv, [wb + 2])
            wx2 = plsc.load_gather(allv, [wb + 3])
            wy2 = plsc.load_gather(allv, [wb + 4])
            wa = plsc.load_gather(allv, [wb + 5])
            wcls = plsc.load_gather(allv, [wb + 6])
            validv = gms > 0.0
            vfv = jnp.where(validv, 1.0, 0.0)

            @plsc.parallel_loop(0, nch, 1, carry=(neg2, zero_i))
            def sup_i(i, cc):
                bv, bpv = cc
                for k in range(4):
                    sl = pl.ds(i * 64 + k * _L, _L)
                    ix1 = jnp.maximum(wx1, x1v[sl])
                    iy1 = jnp.maximum(wy1, y1v[sl])
                    ix2 = jnp.minimum(wx2, x2v[sl])
                    iy2 = jnp.minimum(wy2, y2v[sl])
                    inter = (jnp.maximum(ix2 - ix1, 0.0)
                             * jnp.maximum(iy2 - iy1, 0.0))
                    iou = inter / (wa + areav[sl] - inter + 1e-9)
                    s_new = jnp.where(iou > _IOU_THR, -1.0, sv[sl])
                    sv[sl] = s_new
                    pos = i * 64 + k * _L + lane
                    better = s_new > bv
                    bv = jnp.where(better, s_new, bv)
                    bpv = jnp.where(better, pos, bpv)
                return bv, bpv

            @pl.when(tid == 0)
            def _():
                ov = vfv
                ov = jnp.where(lane == 0, wx1 * vfv, ov)
                ov = jnp.where(lane == 1, wy1 * vfv, ov)
                ov = jnp.where(lane == 2, wx2 * vfv, ov)
                ov = jnp.where(lane == 3, wy2 * vfv, ov)
                ov = jnp.where(lane == 4, jnp.where(validv, wcls, -1.0), ov)
                ov = jnp.where(lane == 5, jnp.where(validv, gms, 0.0), ov)
                plsc.store_scatter(outv, [t * 8 + lane], ov, mask=lane < 7)

            return sup_i

        lax.fori_loop(0, _MAX_DET, step, (bestv0, bposv0))

        @pl.when(tid == 0)
        def _():
            pltpu.sync_copy(outv, out_h)

    call = pl.kernel(
        body,
        out_type=jax.ShapeDtypeStruct((_OUT_PAD * 8,), jnp.float32),
        mesh=mesh,
        compiler_params=pltpu.CompilerParams(needs_layout_passes=False),
        scratch_types=[
            pltpu.VMEM((_CH,), jnp.float32),
            pltpu.VMEM((_CH,), jnp.float32),
            pltpu.VMEM((_CH,), jnp.float32),
            pltpu.VMEM((_CH,), jnp.float32),
            pltpu.VMEM((_CH,), jnp.float32),
            pltpu.VMEM((_CH,), jnp.float32),
            pltpu.VMEM((_CH,), jnp.float32),
            pltpu.VMEM((_L,), jnp.float32),
            pltpu.VMEM((_XW,), jnp.float32),
            pltpu.VMEM((_OUT_PAD * 8,), jnp.float32),
            pltpu.VMEM_SHARED((2 * _XW,), jnp.float32),
        ],
    )
    return call(x1, y1, x2, y2, s, c)


def kernel(boxes, scores, classes):
    pad = _NPAD - _N_BOXES
    x1 = jnp.pad(boxes[:, 0], (0, pad))
    y1 = jnp.pad(boxes[:, 1], (0, pad))
    x2 = jnp.pad(boxes[:, 2], (0, pad))
    y2 = jnp.pad(boxes[:, 3], (0, pad))
    s = jnp.pad(scores, (0, pad), constant_values=-1.0)
    c = jnp.pad(classes.astype(jnp.float32), (0, pad))

    out = _nms_call(x1, y1, x2, y2, s, c)
    out = out.reshape(_OUT_PAD, 8)[:_MAX_DET]
    boxes_b = out[:, 0:4]
    labels_b = out[:, 4].astype(jnp.int32)
    scores_b = out[:, 5]
    sel_valid = out[:, 6] > 0.0
    return boxes_b, labels_b, scores_b, sel_valid
